# baseline (speedup 1.0000x reference)
import functools
import math

import jax
import jax.numpy as jnp
from jax import lax
from jax.experimental import pallas as pl
from jax.experimental.pallas import tpu as pltpu
from jax.experimental.pallas import tpu_sc as plsc

_VOCAB = 1000000
_EMB = 64
_B = 4096
_L = 200

_info = plsc.get_sparse_core_info()
_NC = _info.num_cores
_NS = _info.num_subcores
_NW = _NC * _NS
_CHUNK = 128
_CPW = _L
_NBUF = 4

_SCALE = math.sqrt(_EMB)

_LT = _L // 8
_ET = _EMB // 8
_TV = _VOCAB // _CHUNK
_TPW = _TV // _NW
_TREM = _TV - _TPW * _NW
_VTAIL = _VOCAB - _TV * _CHUNK

_mesh = plsc.VectorSubcoreMesh(core_axis_name="c", subcore_axis_name="s")


def _relayout_body(tabt_hbm, tail_hbm, t2_hbm, tin, tout, ttmp, isem, osem):
    wid = lax.axis_index("s") * _NC + lax.axis_index("c")
    iota = lax.iota(jnp.int32, 16)
    base = wid * _TPW

    def istart(i, b):
        pltpu.async_copy(
            tabt_hbm.at[:, pl.ds((base + i) * _CHUNK, _CHUNK)], tin[b], isem[b]
        )

    def iwait(i, b):
        pltpu.make_async_copy(
            tabt_hbm.at[:, pl.ds((base + i) * _CHUNK, _CHUNK)], tin[b], isem[b]
        ).wait()

    def ostart(i, b):
        pltpu.async_copy(
            tout[b], t2_hbm.at[pl.ds((base + i) * _CHUNK, _CHUNK)], osem[b]
        )

    def owait(i, b):
        pltpu.make_async_copy(
            tout[b], t2_hbm.at[pl.ds((base + i) * _CHUNK, _CHUNK)], osem[b]
        ).wait()

    def trans(b, nk):
        @plsc.parallel_loop(0, nk, unroll=8)
        def _(k):
            col_idx = jnp.full((16,), 0, jnp.int32) + k
            for seg in range(_EMB // 16):
                row_idx = seg * 16 + iota
                v = plsc.load_gather(tin[b], [row_idx, col_idx])
                tout[b][k, pl.ds(seg * 16, 16)] = v * _SCALE

    istart(0, 0)
    istart(1, 1)
    for b in range(_NBUF):
        i = b
        if i >= 2:
            owait(i - 2, (b - 2) % _NBUF)
        istart(i + 2, (b + 2) % _NBUF)
        iwait(i, b)
        trans(b, _CHUNK)
        ostart(i, b)

    @pl.loop(0, (_TPW - 2 * _NBUF) // _NBUF)
    def _(r):
        i0 = _NBUF + r * _NBUF
        for b in range(_NBUF):
            i = i0 + b
            owait(i - 2, (b - 2) % _NBUF)
            istart(i + 2, (b + 2) % _NBUF)
            iwait(i, b)
            trans(b, _CHUNK)
            ostart(i, b)

    for b in range(_NBUF):
        i = _TPW - _NBUF + b
        owait(i - 2, (b - 2) % _NBUF)
        if i + 2 < _TPW:
            istart(i + 2, (b + 2) % _NBUF)
        iwait(i, b)
        trans(b, _CHUNK)
        ostart(i, b)

    owait(_TPW - 2, (_NBUF - 2) % _NBUF)
    owait(_TPW - 1, _NBUF - 1)

    @pl.when(wid < _TREM)
    def _():
        vt = _TV - _TREM + wid
        pltpu.sync_copy(tabt_hbm.at[:, pl.ds(vt * _CHUNK, _CHUNK)], tin[0])
        trans(0, _CHUNK)
        pltpu.sync_copy(tout[0], t2_hbm.at[pl.ds(vt * _CHUNK, _CHUNK)])

    @pl.when(wid == _TREM)
    def _():
        pltpu.sync_copy(tail_hbm, ttmp)
        pltpu.sync_copy(ttmp, t2_hbm.at[pl.ds(_TV * _CHUNK, _VTAIL)])


_relayout = functools.partial(
    pl.kernel,
    mesh=_mesh,
    out_type=jax.ShapeDtypeStruct((_VOCAB, _EMB), jnp.float32),
    scratch_types=[
        [pltpu.VMEM((_EMB, _CHUNK), jnp.float32) for _ in range(_NBUF)],
        [pltpu.VMEM((_CHUNK, _EMB), jnp.float32) for _ in range(_NBUF)],
        pltpu.VMEM((_VTAIL, _EMB), jnp.float32),
        [pltpu.SemaphoreType.DMA for _ in range(_NBUF)],
        [pltpu.SemaphoreType.DMA for _ in range(_NBUF)],
    ],
    compiler_params=pltpu.CompilerParams(
        use_tc_tiling_on_sc=True, needs_layout_passes=False
    ),
)(_relayout_body)


def _gather_body(tok_hbm, table_hbm, out_hbm, idx_v, rows, buf, gsem, osem):
    wid = lax.axis_index("s") * _NC + lax.axis_index("c")
    pltpu.sync_copy(tok_hbm.at[:, wid], idx_v)
    iota = lax.iota(jnp.int32, 16)

    def gstart(j, b):
        pltpu.async_copy(table_hbm.at[idx_v.at[j // 8, j % 8]], rows[b], gsem[b])

    def gwait(j, b):
        pltpu.make_async_copy(
            table_hbm.at[idx_v.at[j // 8, j % 8]], rows[b], gsem[b]
        ).wait()

    def ostart(j, b):
        pltpu.async_copy(buf[b], out_hbm.at[pl.ds(j * _ET, _ET), wid], osem[b])

    def owait(j, b):
        pltpu.make_async_copy(
            buf[b], out_hbm.at[pl.ds(j * _ET, _ET), wid], osem[b]
        ).wait()

    def transpose(b):
        @plsc.parallel_loop(0, _EMB, unroll=8)
        def _(e):
            col_idx = jnp.full((16,), 0, jnp.int32) + e
            for kg in range(_CHUNK // 16):
                row_idx = kg * 16 + iota
                v = plsc.load_gather(rows[b], [row_idx, col_idx])
                buf[b][e // 8, e % 8, pl.ds(kg * 16, 16)] = v

    gstart(0, 0)
    gstart(1, 1)
    for b in range(_NBUF):
        j = b
        if j >= 2:
            owait(j - 2, (b - 2) % _NBUF)
        gstart(j + 2, (b + 2) % _NBUF)
        gwait(j, b)
        transpose(b)
        ostart(j, b)

    @pl.loop(0, (_CPW - 2 * _NBUF) // _NBUF)
    def _(r):
        j0 = _NBUF + r * _NBUF
        for b in range(_NBUF):
            j = j0 + b
            owait(j - 2, (b - 2) % _NBUF)
            gstart(j + 2, (b + 2) % _NBUF)
            gwait(j, b)
            transpose(b)
            ostart(j, b)

    for b in range(_NBUF):
        j = _CPW - _NBUF + b
        owait(j - 2, (b - 2) % _NBUF)
        if j + 2 < _CPW:
            gstart(j + 2, (b + 2) % _NBUF)
        gwait(j, b)
        transpose(b)
        ostart(j, b)

    owait(_CPW - 2, (_NBUF - 2) % _NBUF)
    owait(_CPW - 1, _NBUF - 1)


_gather = functools.partial(
    pl.kernel,
    mesh=_mesh,
    out_type=jax.ShapeDtypeStruct((_L * _ET, _NW, 8, _CHUNK), jnp.float32),
    scratch_types=[
        pltpu.VMEM((_LT, 8, _CHUNK), jnp.int32),
        [pltpu.VMEM((_CHUNK, _EMB), jnp.float32) for _ in range(_NBUF)],
        [pltpu.VMEM((_ET, 8, _CHUNK), jnp.float32) for _ in range(_NBUF)],
        [pltpu.SemaphoreType.DMA for _ in range(_NBUF)],
        [pltpu.SemaphoreType.DMA for _ in range(_NBUF)],
    ],
    compiler_params=pltpu.CompilerParams(
        use_tc_tiling_on_sc=False, needs_layout_passes=False
    ),
)(_gather_body)


def kernel(tokens, table):
    tail = table[_TV * _CHUNK :] * _SCALE
    t2 = _relayout(table.T, tail)
    tok = (
        tokens.astype(jnp.int32)
        .T.reshape(_LT, 8, _NW, _CHUNK)
        .transpose(0, 2, 1, 3)
    )
    out = _gather(tok, t2)
    return (
        out.reshape(_L, _ET, _NW, 8, _CHUNK)
        .transpose(2, 4, 0, 1, 3)
        .reshape(_B, _L, _EMB)
    )


# speedup vs baseline: 2.2351x; 2.2351x over previous
import functools
import math

import jax
import jax.numpy as jnp
from jax import lax
from jax.experimental import pallas as pl
from jax.experimental.pallas import tpu as pltpu
from jax.experimental.pallas import tpu_sc as plsc

_VOCAB = 1000000
_EMB = 64
_B = 4096
_L = 200

_info = plsc.get_sparse_core_info()
_NC = _info.num_cores
_NS = _info.num_subcores
_NW = _NC * _NS
_CHUNK = 128
_CPW = _L
_NBUF = 4

_SCALE = math.sqrt(_EMB)

_LT = _L // 8
_ET = _EMB // 8
_TV = _VOCAB // _CHUNK
_TPW = _TV // _NW
_TREM = _TV - _TPW * _NW
_VTAIL = _VOCAB - _TV * _CHUNK

_mesh = plsc.VectorSubcoreMesh(core_axis_name="c", subcore_axis_name="s")


def _relayout_body(tabt_hbm, tail_hbm, t2_hbm, tin, tout, ttmp, isem, osem):
    wid = lax.axis_index("s") * _NC + lax.axis_index("c")
    iota = lax.iota(jnp.int32, 16)
    base = wid * _TPW

    def istart(i, b):
        pltpu.async_copy(
            tabt_hbm.at[:, pl.ds((base + i) * _CHUNK, _CHUNK)], tin[b], isem[b]
        )

    def iwait(i, b):
        pltpu.make_async_copy(
            tabt_hbm.at[:, pl.ds((base + i) * _CHUNK, _CHUNK)], tin[b], isem[b]
        ).wait()

    def ostart(i, b):
        pltpu.async_copy(
            tout[b], t2_hbm.at[pl.ds((base + i) * _CHUNK, _CHUNK)], osem[b]
        )

    def owait(i, b):
        pltpu.make_async_copy(
            tout[b], t2_hbm.at[pl.ds((base + i) * _CHUNK, _CHUNK)], osem[b]
        ).wait()

    def trans(b, nk):
        @plsc.parallel_loop(0, nk, unroll=8)
        def _(k):
            col_idx = jnp.full((16,), 0, jnp.int32) + k
            for seg in range(_EMB // 16):
                row_idx = seg * 16 + iota
                v = plsc.load_gather(tin[b], [row_idx, col_idx])
                tout[b][k, pl.ds(seg * 16, 16)] = v * _SCALE

    istart(0, 0)
    istart(1, 1)
    for b in range(_NBUF):
        i = b
        if i >= 2:
            owait(i - 2, (b - 2) % _NBUF)
        istart(i + 2, (b + 2) % _NBUF)
        iwait(i, b)
        trans(b, _CHUNK)
        ostart(i, b)

    @pl.loop(0, (_TPW - 2 * _NBUF) // _NBUF)
    def _(r):
        i0 = _NBUF + r * _NBUF
        for b in range(_NBUF):
            i = i0 + b
            owait(i - 2, (b - 2) % _NBUF)
            istart(i + 2, (b + 2) % _NBUF)
            iwait(i, b)
            trans(b, _CHUNK)
            ostart(i, b)

    for b in range(_NBUF):
        i = _TPW - _NBUF + b
        owait(i - 2, (b - 2) % _NBUF)
        if i + 2 < _TPW:
            istart(i + 2, (b + 2) % _NBUF)
        iwait(i, b)
        trans(b, _CHUNK)
        ostart(i, b)

    owait(_TPW - 2, (_NBUF - 2) % _NBUF)
    owait(_TPW - 1, _NBUF - 1)

    @pl.when(wid < _TREM)
    def _():
        vt = _TV - _TREM + wid
        pltpu.sync_copy(tabt_hbm.at[:, pl.ds(vt * _CHUNK, _CHUNK)], tin[0])
        trans(0, _CHUNK)
        pltpu.sync_copy(tout[0], t2_hbm.at[pl.ds(vt * _CHUNK, _CHUNK)])

    @pl.when(wid == _TREM)
    def _():
        pltpu.sync_copy(tail_hbm, ttmp)
        pltpu.sync_copy(ttmp, t2_hbm.at[pl.ds(_TV * _CHUNK, _VTAIL)])


_relayout = functools.partial(
    pl.kernel,
    mesh=_mesh,
    out_type=jax.ShapeDtypeStruct((_VOCAB, _EMB), jnp.float32),
    scratch_types=[
        [pltpu.VMEM((_EMB, _CHUNK), jnp.float32) for _ in range(_NBUF)],
        [pltpu.VMEM((_CHUNK, _EMB), jnp.float32) for _ in range(_NBUF)],
        pltpu.VMEM((_VTAIL, _EMB), jnp.float32),
        [pltpu.SemaphoreType.DMA for _ in range(_NBUF)],
        [pltpu.SemaphoreType.DMA for _ in range(_NBUF)],
    ],
    compiler_params=pltpu.CompilerParams(
        use_tc_tiling_on_sc=True, needs_layout_passes=False
    ),
)(_relayout_body)


def _gather_body(tok_hbm, table_hbm, out_hbm, idx_v, rows, buf, gsem, osem):
    wid = lax.axis_index("s") * _NC + lax.axis_index("c")
    pltpu.sync_copy(tok_hbm.at[:, wid], idx_v)
    iota = lax.iota(jnp.int32, 16)

    def gstart(j, b):
        pltpu.async_copy(table_hbm.at[idx_v.at[j // 8, j % 8]], rows[b], gsem[b])

    def gwait(j, b):
        pltpu.make_async_copy(
            table_hbm.at[idx_v.at[j // 8, j % 8]], rows[b], gsem[b]
        ).wait()

    def ostart(j, b):
        pltpu.async_copy(buf[b], out_hbm.at[pl.ds(j * _ET, _ET), wid], osem[b])

    def owait(j, b):
        pltpu.make_async_copy(
            buf[b], out_hbm.at[pl.ds(j * _ET, _ET), wid], osem[b]
        ).wait()

    skew = [jnp.bitwise_and(iota + r, 15) for r in range(16)]

    def transpose(b):
        @plsc.parallel_loop(0, _CHUNK // 16)
        def _(kg):
            row_idx = kg * 16 + iota

            @plsc.parallel_loop(0, _EMB, step=16)
            def _(e0):
                for r in range(16):
                    col = skew[r] + e0
                    v = plsc.load_gather(rows[b], [row_idx, col])
                    plsc.store_scatter(
                        buf[b],
                        [
                            jax.lax.shift_right_logical(col, 3),
                            jnp.bitwise_and(col, 7),
                            row_idx,
                        ],
                        v * _SCALE,
                    )

    gstart(0, 0)
    gstart(1, 1)
    for b in range(_NBUF):
        j = b
        if j >= 2:
            owait(j - 2, (b - 2) % _NBUF)
        gstart(j + 2, (b + 2) % _NBUF)
        gwait(j, b)
        transpose(b)
        ostart(j, b)

    @pl.loop(0, (_CPW - 2 * _NBUF) // _NBUF)
    def _(r):
        j0 = _NBUF + r * _NBUF
        for b in range(_NBUF):
            j = j0 + b
            owait(j - 2, (b - 2) % _NBUF)
            gstart(j + 2, (b + 2) % _NBUF)
            gwait(j, b)
            transpose(b)
            ostart(j, b)

    for b in range(_NBUF):
        j = _CPW - _NBUF + b
        owait(j - 2, (b - 2) % _NBUF)
        if j + 2 < _CPW:
            gstart(j + 2, (b + 2) % _NBUF)
        gwait(j, b)
        transpose(b)
        ostart(j, b)

    owait(_CPW - 2, (_NBUF - 2) % _NBUF)
    owait(_CPW - 1, _NBUF - 1)


_gather = functools.partial(
    pl.kernel,
    mesh=_mesh,
    out_type=jax.ShapeDtypeStruct((_L * _ET, _NW, 8, _CHUNK), jnp.float32),
    scratch_types=[
        pltpu.VMEM((_LT, 8, _CHUNK), jnp.int32),
        [pltpu.VMEM((_CHUNK, _EMB), jnp.float32) for _ in range(_NBUF)],
        [pltpu.VMEM((_ET, 8, _CHUNK), jnp.float32) for _ in range(_NBUF)],
        [pltpu.SemaphoreType.DMA for _ in range(_NBUF)],
        [pltpu.SemaphoreType.DMA for _ in range(_NBUF)],
    ],
    compiler_params=pltpu.CompilerParams(
        use_tc_tiling_on_sc=False, needs_layout_passes=False
    ),
)(_gather_body)


def kernel(tokens, table):
    tok = (
        tokens.astype(jnp.int32)
        .T.reshape(_LT, 8, _NW, _CHUNK)
        .transpose(0, 2, 1, 3)
    )
    out = _gather(tok, table)
    return (
        out.reshape(_L, _ET, _NW, 8, _CHUNK)
        .transpose(2, 4, 0, 1, 3)
        .reshape(_B, _L, _EMB)
    )


# speedup vs baseline: 3.1392x; 1.4045x over previous
import functools
import math

import jax
import jax.numpy as jnp
from jax import lax
from jax.experimental import pallas as pl
from jax.experimental.pallas import tpu as pltpu
from jax.experimental.pallas import tpu_sc as plsc

_VOCAB = 1000000
_EMB = 64
_B = 4096
_L = 200

_info = plsc.get_sparse_core_info()
_NC = _info.num_cores
_NS = _info.num_subcores
_NW = _NC * _NS
_CHUNK = 128
_CPW = _L
_NBUF = 4

_SCALE = math.sqrt(_EMB)

_LT = _L // 8
_ET = _EMB // 8
_TV = _VOCAB // _CHUNK
_TPW = _TV // _NW
_TREM = _TV - _TPW * _NW
_VTAIL = _VOCAB - _TV * _CHUNK


def _body(tabt_hbm, tail_hbm, tok_hbm, out_hbm, t2_hbm,
          tin, tout, idx_v, idxrow, rows, buf, isem, o2sem, gsem, osem,
          barsem):
    wid = lax.axis_index("s") * _NC + lax.axis_index("c")
    iota = lax.iota(jnp.int32, 16)
    skew = [jnp.bitwise_and(iota + r, 15) for r in range(16)]

    pltpu.sync_copy(tok_hbm.at[:, wid], idx_v)

    base = wid * _TPW

    def istart(i, b):
        pltpu.async_copy(
            tabt_hbm.at[:, pl.ds((base + i) * _CHUNK, _CHUNK)], tin[b], isem[b]
        )

    def iwait(i, b):
        pltpu.make_async_copy(
            tabt_hbm.at[:, pl.ds((base + i) * _CHUNK, _CHUNK)], tin[b], isem[b]
        ).wait()

    def o2start(i, b):
        pltpu.async_copy(
            tout[b], t2_hbm.at[pl.ds((base + i) * _EMB, _EMB)], o2sem[b]
        )

    def o2wait(i, b):
        pltpu.make_async_copy(
            tout[b], t2_hbm.at[pl.ds((base + i) * _EMB, _EMB)], o2sem[b]
        ).wait()

    def trans_a(b):
        @plsc.parallel_loop(0, _CHUNK // 16)
        def _(kg):
            k_idx = kg * 16 + iota
            kh = lax.shift_right_logical(k_idx, 1)
            kb = lax.shift_left(jnp.bitwise_and(k_idx, 1), 6)

            @plsc.parallel_loop(0, _EMB, step=16)
            def _(e0):
                for r in range(16):
                    col = skew[r] + e0
                    v = plsc.load_gather(tin[b], [col, k_idx])
                    plsc.store_scatter(tout[b], [kh, kb + col], v * _SCALE)

    istart(0, 0)
    istart(1, 1)
    for b in range(2):
        i = b
        iwait(i, b)
        trans_a(b)
        istart(i + 2, b)
        o2start(i, b)

    @pl.loop(0, (_TPW - 4) // 2)
    def _(r):
        i0 = 2 + 2 * r
        for b in range(2):
            i = i0 + b
            o2wait(i - 2, b)
            iwait(i, b)
            trans_a(b)
            istart(i + 2, b)
            o2start(i, b)

    for b in range(2):
        i = _TPW - 2 + b
        o2wait(i - 2, b)
        iwait(i, b)
        trans_a(b)
        o2start(i, b)
    o2wait(_TPW - 2, 0)
    o2wait(_TPW - 1, 1)

    @pl.when(wid < _TREM)
    def _():
        vt = _TV - _TREM + wid
        pltpu.sync_copy(tabt_hbm.at[:, pl.ds(vt * _CHUNK, _CHUNK)], tin[0])
        trans_a(0)
        pltpu.sync_copy(tout[0], t2_hbm.at[pl.ds(vt * _EMB, _EMB)])

    @pl.when(wid == _TREM)
    def _():
        nr = _VTAIL // 2
        pltpu.sync_copy(tail_hbm, tout[0].at[pl.ds(0, nr)])
        pltpu.sync_copy(
            tout[0].at[pl.ds(0, nr)], t2_hbm.at[pl.ds(_TV * _EMB, nr)]
        )

    plsc.subcore_barrier()

    @pl.when(lax.axis_index("s") == 0)
    def _():
        pltpu.core_barrier(barsem, core_axis_name="c")

    plsc.subcore_barrier()

    def prep(j, rb):
        for g in range(8):
            idxrow[rb][pl.ds(g * 16, 16)] = lax.shift_right_logical(
                idx_v[j // 8, j % 8, pl.ds(g * 16, 16)], 1
            )

    def gstart(j, rb):
        pltpu.async_copy(t2_hbm.at[idxrow[rb]], rows[rb], gsem[rb])

    def gwait(j, rb):
        pltpu.make_async_copy(t2_hbm.at[idxrow[rb]], rows[rb], gsem[rb]).wait()

    def ostart(j, b):
        pltpu.async_copy(buf[b], out_hbm.at[pl.ds(j * _ET, _ET), wid], osem[b])

    def owait(j, b):
        pltpu.make_async_copy(
            buf[b], out_hbm.at[pl.ds(j * _ET, _ET), wid], osem[b]
        ).wait()

    def trans_b(j, rb, b):
        @plsc.parallel_loop(0, _CHUNK // 16)
        def _(kg):
            k_idx = kg * 16 + iota
            par = lax.shift_left(
                jnp.bitwise_and(idx_v[j // 8, j % 8, pl.ds(kg * 16, 16)], 1), 6
            )

            @plsc.parallel_loop(0, _EMB, step=16)
            def _(e0):
                for r in range(16):
                    col = skew[r] + e0
                    v = plsc.load_gather(rows[rb], [k_idx, col + par])
                    plsc.store_scatter(
                        buf[b],
                        [
                            lax.shift_right_logical(col, 3),
                            jnp.bitwise_and(col, 7),
                            k_idx,
                        ],
                        v,
                    )

    prep(0, 0)
    gstart(0, 0)
    prep(1, 1)
    gstart(1, 1)
    for b in range(_NBUF):
        j = b
        if j >= 2:
            owait(j - 2, (b - 2) % _NBUF)
        gwait(j, j % 2)
        trans_b(j, j % 2, b)
        prep(j + 2, j % 2)
        gstart(j + 2, j % 2)
        ostart(j, b)

    @pl.loop(0, (_CPW - 2 * _NBUF) // _NBUF)
    def _(r):
        j0 = _NBUF + r * _NBUF
        for b in range(_NBUF):
            j = j0 + b
            owait(j - 2, (b - 2) % _NBUF)
            gwait(j, b % 2)
            trans_b(j, b % 2, b)
            prep(j + 2, b % 2)
            gstart(j + 2, b % 2)
            ostart(j, b)

    for b in range(_NBUF):
        j = _CPW - _NBUF + b
        owait(j - 2, (b - 2) % _NBUF)
        gwait(j, b % 2)
        trans_b(j, b % 2, b)
        if j + 2 < _CPW:
            prep(j + 2, b % 2)
            gstart(j + 2, b % 2)
        ostart(j, b)

    owait(_CPW - 2, (_NBUF - 2) % _NBUF)
    owait(_CPW - 1, _NBUF - 1)


_mesh = plsc.VectorSubcoreMesh(core_axis_name="c", subcore_axis_name="s")

_fused = functools.partial(
    pl.kernel,
    mesh=_mesh,
    out_type=[
        jax.ShapeDtypeStruct((_L * _ET, _NW, 8, _CHUNK), jnp.float32),
        jax.ShapeDtypeStruct((_VOCAB // 2, _CHUNK), jnp.float32),
    ],
    scratch_types=[
        [pltpu.VMEM((_EMB, _CHUNK), jnp.float32) for _ in range(2)],
        [pltpu.VMEM((_EMB, _CHUNK), jnp.float32) for _ in range(2)],
        pltpu.VMEM((_LT, 8, _CHUNK), jnp.int32),
        [pltpu.VMEM((_CHUNK,), jnp.int32) for _ in range(2)],
        [pltpu.VMEM((_CHUNK, _CHUNK), jnp.float32) for _ in range(2)],
        [pltpu.VMEM((_ET, 8, _CHUNK), jnp.float32) for _ in range(_NBUF)],
        [pltpu.SemaphoreType.DMA for _ in range(2)],
        [pltpu.SemaphoreType.DMA for _ in range(2)],
        [pltpu.SemaphoreType.DMA for _ in range(2)],
        [pltpu.SemaphoreType.DMA for _ in range(_NBUF)],
        pltpu.SemaphoreType.REGULAR,
    ],
    compiler_params=pltpu.CompilerParams(
        use_tc_tiling_on_sc=True, needs_layout_passes=False
    ),
)(_body)


def kernel(tokens, table):
    tail = (table[_TV * _CHUNK :] * _SCALE).reshape(_VTAIL // 2, _CHUNK)
    tok = (
        tokens.astype(jnp.int32)
        .T.reshape(_LT, 8, _NW, _CHUNK)
        .transpose(0, 2, 1, 3)
    )
    out, _ = _fused(table.T, tail, tok)
    return (
        out.reshape(_L, _ET, _NW, 8, _CHUNK)
        .transpose(2, 4, 0, 1, 3)
        .reshape(_B, _L, _EMB)
    )
